# Initial kernel scaffold; baseline (speedup 1.0000x reference)
import jax
import jax.numpy as jnp
from jax.experimental import pallas as pl


def kernel(x, xyz):
    raise NotImplementedError("write your pallas kernel here")


# speedup vs baseline: 16.5301x; 16.5301x over previous
import functools

import jax
import jax.numpy as jnp
from jax import lax
from jax.experimental import pallas as pl
from jax.experimental.pallas import tpu as pltpu
from jax.experimental.pallas import tpu_sc as plsc

N_SAMPLE = 2048
KNN = 4
B, D, N = 4, 128, 8192


def _fps_body(xyz_ref, ind_ref, sxyz_ref, dist_ref):
    px = xyz_ref[:, 0, :]
    py = xyz_ref[:, 1, :]
    pz = xyz_ref[:, 2, :]
    iota_n = lax.broadcasted_iota(jnp.int32, (B, N), 1)
    dist_ref[...] = jnp.full((B, N), jnp.inf, dtype=jnp.float32)

    def step(i, far):
        ind_ref[pl.ds(i, 1), :] = far.reshape(1, B)
        sel = iota_n == far
        cx = jnp.sum(jnp.where(sel, px, 0.0), axis=1, keepdims=True)
        cy = jnp.sum(jnp.where(sel, py, 0.0), axis=1, keepdims=True)
        cz = jnp.sum(jnp.where(sel, pz, 0.0), axis=1, keepdims=True)
        sxyz_ref[pl.ds(i, 1), :, :] = jnp.concatenate(
            [cx, cy, cz], axis=1).T.reshape(1, 3, B)
        dx = px - cx
        dy = py - cy
        dz = pz - cz
        d = (dx * dx + dy * dy) + dz * dz
        dist = jnp.minimum(dist_ref[...], d)
        dist_ref[...] = dist
        mx = jnp.max(dist, axis=1, keepdims=True)
        far_new = jnp.min(
            jnp.where(dist == mx, iota_n, N), axis=1, keepdims=True
        ).astype(jnp.int32)
        return far_new

    lax.fori_loop(0, N_SAMPLE, step, jnp.zeros((B, 1), jnp.int32))


def _run_fps(xyz):
    return pl.pallas_call(
        _fps_body,
        out_shape=(
            jax.ShapeDtypeStruct((N_SAMPLE, B), jnp.int32),
            jax.ShapeDtypeStruct((N_SAMPLE, 3, B), jnp.float32),
        ),
        scratch_shapes=[pltpu.VMEM((B, N), jnp.float32)],
    )(xyz)


_BM = 256
_NCH = 2048


def _bf16_rne(v):
    u = lax.bitcast_convert_type(v, jnp.uint32)
    lsb = (u >> 16) & jnp.uint32(1)
    r = (u + jnp.uint32(0x7FFF) + lsb) & jnp.uint32(0xFFFF0000)
    return lax.bitcast_convert_type(r, jnp.float32)


def _knn_body(xyzt_ref, sxyz_ref, nbr_ref):
    sx = sxyz_ref[0, 0, :].reshape(1, _BM)
    sy = sxyz_ref[0, 1, :].reshape(1, _BM)
    sz = sxyz_ref[0, 2, :].reshape(1, _BM)
    a2 = (sx * sx + sy * sy) + sz * sz
    sxr = _bf16_rne(sx)
    syr = _bf16_rne(sy)
    szr = _bf16_rne(sz)

    big = jnp.float32(jnp.inf)
    best_v = [jnp.full((1, _BM), big, jnp.float32) for _ in range(KNN)]
    best_i = [jnp.full((1, _BM), N, jnp.int32) for _ in range(KNN)]

    for c in range(N // _NCH):
        n0 = c * _NCH
        px = xyzt_ref[0, pl.ds(n0, _NCH), 0].reshape(_NCH, 1)
        py = xyzt_ref[0, pl.ds(n0, _NCH), 1].reshape(_NCH, 1)
        pz = xyzt_ref[0, pl.ds(n0, _NCH), 2].reshape(_NCH, 1)
        b2 = (px * px + py * py) + pz * pz
        pxr = _bf16_rne(px)
        pyr = _bf16_rne(py)
        pzr = _bf16_rne(pz)
        ab = (pxr * sxr + pyr * syr) + pzr * szr
        d2 = jnp.sqrt(jnp.maximum((a2 + b2) - 2.0 * ab, 0.0))
        iota = lax.broadcasted_iota(jnp.int32, (_NCH, _BM), 0) + n0
        for _ in range(KNN):
            mn = jnp.min(d2, axis=0, keepdims=True)
            arg = jnp.min(
                jnp.where(d2 == mn, iota, N), axis=0, keepdims=True)
            cv, ci = mn, arg
            for k in range(KNN):
                take = cv < best_v[k]
                nv = jnp.where(take, cv, best_v[k])
                ni = jnp.where(take, ci, best_i[k])
                cv = jnp.where(take, best_v[k], cv)
                ci = jnp.where(take, best_i[k], ci)
                best_v[k], best_i[k] = nv, ni
            d2 = jnp.where(iota == arg, big, d2)

    for k in range(KNN):
        nbr_ref[0, k, :] = best_i[k][0, :]


def _run_knn(xyz_t, sxyz0):
    grid = (B, N_SAMPLE // _BM)
    return pl.pallas_call(
        _knn_body,
        grid=grid,
        in_specs=[
            pl.BlockSpec((1, N, 3), lambda b, m: (b, 0, 0)),
            pl.BlockSpec((1, 3, _BM), lambda b, m: (b, 0, m)),
        ],
        out_specs=pl.BlockSpec((1, KNN, _BM), lambda b, m: (b, 0, m)),
        out_shape=jax.ShapeDtypeStruct((B, KNN, N_SAMPLE), jnp.int32),
    )(xyz_t, sxyz0)


_XW = 16


def _gather_body(xt_ref, xyzp_ref, nbr_ref, sx_ref, sxyz_ref,
                 idx_v, rows_v, xrows_v, out_v, oxyz_v, sem1, sem2):
    info = plsc.get_sparse_core_info()
    nw = info.num_cores * info.num_subcores
    wid = lax.axis_index("s") * info.num_cores + lax.axis_index("c")
    rows_total = B * N_SAMPLE
    rows_per_w = rows_total // nw
    m_chunk = 32
    n_chunks = rows_per_w // m_chunk
    base_row = wid * rows_per_w
    batch = base_row // N_SAMPLE
    n_off = batch * N

    def do_chunk(ci, _):
        row0 = base_row + ci * m_chunk
        pltpu.sync_copy(nbr_ref.at[pl.ds(row0 * KNN, m_chunk * KNN)], idx_v)
        for j in range(m_chunk * KNN // 16):
            sl = pl.ds(j * 16, 16)
            idx_v[sl] = idx_v[sl] + n_off
        pltpu.async_copy(xt_ref.at[idx_v], rows_v, sem1).wait()
        pltpu.async_copy(xyzp_ref.at[idx_v], xrows_v, sem2).wait()
        quarter = jnp.float32(0.25)
        for m in range(m_chunk):
            r = m * KNN
            for j in range(D // 16):
                sl = pl.ds(j * 16, 16)
                v = jnp.maximum(
                    jnp.maximum(rows_v[r, sl], rows_v[r + 1, sl]),
                    jnp.maximum(rows_v[r + 2, sl], rows_v[r + 3, sl]))
                out_v[m, sl] = v
            sl0 = pl.ds(0, _XW)
            s = ((xrows_v[r, sl0] + xrows_v[r + 1, sl0])
                 + (xrows_v[r + 2, sl0] + xrows_v[r + 3, sl0]))
            oxyz_v[m, :] = s * quarter
        pltpu.sync_copy(out_v, sx_ref.at[pl.ds(row0, m_chunk)])
        pltpu.sync_copy(oxyz_v, sxyz_ref.at[pl.ds(row0, m_chunk)])
        return ()

    lax.fori_loop(0, n_chunks, do_chunk, ())


def _run_gather(xt, xyzp, nbr_flat):
    mesh = plsc.VectorSubcoreMesh(core_axis_name="c", subcore_axis_name="s")
    kfn = pl.kernel(
        _gather_body,
        mesh=mesh,
        out_type=(
            jax.ShapeDtypeStruct((B * N_SAMPLE, D), jnp.float32),
            jax.ShapeDtypeStruct((B * N_SAMPLE, _XW), jnp.float32),
        ),
        scratch_types=[
            pltpu.VMEM((32 * KNN,), jnp.int32),
            pltpu.VMEM((32 * KNN, D), jnp.float32),
            pltpu.VMEM((32 * KNN, D), jnp.float32),
            pltpu.VMEM((32, D), jnp.float32),
            pltpu.VMEM((32, _XW), jnp.float32),
            pltpu.SemaphoreType.DMA,
            pltpu.SemaphoreType.DMA,
        ],
    )
    return kfn(xt, xyzp, nbr_flat)


@jax.jit
def kernel(x, xyz):
    ind_t, sxyz_t = _run_fps(xyz)
    sample_ind = ind_t.T
    sxyz0 = jnp.transpose(sxyz_t, (2, 1, 0))
    xyz_t = jnp.transpose(xyz, (0, 2, 1))
    nbr = _run_knn(xyz_t, sxyz0)
    neighbor_ind = jnp.transpose(nbr, (0, 2, 1))

    xt = jnp.transpose(x, (0, 2, 1)).reshape(B * N, D)
    xyzp = jnp.concatenate(
        [xyz_t, jnp.zeros((B, N, D - 3), jnp.float32)], axis=2
    ).reshape(B * N, D)
    nbr_flat = neighbor_ind.reshape(B * N_SAMPLE * KNN)
    sx_rows, sxyz_rows = _run_gather(xt, xyzp, nbr_flat)
    sample_x = jnp.transpose(
        sx_rows.reshape(B, N_SAMPLE, D), (0, 2, 1))
    sample_xyz = jnp.transpose(
        sxyz_rows.reshape(B, N_SAMPLE, _XW)[:, :, :3], (0, 2, 1))
    return (sample_x, sample_xyz, sample_ind, neighbor_ind)
